# Initial kernel scaffold; baseline (speedup 1.0000x reference)
import jax
import jax.numpy as jnp
from jax.experimental import pallas as pl


def kernel(x, edge_index, W_ih0, W_hh0, b_ih0, b_hh0, W_ih1, W_hh1, b_ih1, b_hh1, Wg1, bg1, Wg2, bg2, Wg3, bg3):
    raise NotImplementedError("write your pallas kernel here")


# speedup vs baseline: 10.6020x; 10.6020x over previous
import functools

import jax
import jax.numpy as jnp
from jax import lax
from jax.experimental import pallas as pl
from jax.experimental.pallas import tpu as pltpu
from jax.experimental.pallas import tpu_sc as plsc

N = 10000
T = 12
IN_DIM = 2
H = 128
OUT_DIM = 12
E = 320000

NC = 2
NS = 16
HALF = N // NC
TRASH = 512
ACC_R = HALF + TRASH
EPS = E // NS
K = 80
CH = EPS // K
RPS = 344
RPS_TAIL = ACC_R - NS * RPS
RPW = 312
RPW_TAIL = HALF - NS * RPW


def _acc_setup(zeros_hbm, acc, s):
    pltpu.sync_copy(zeros_hbm.at[pl.ds(s * RPS, RPS)],
                    acc.at[pl.ds(s * RPS, RPS)])

    @pl.when(s == NS - 1)
    def _():
        pltpu.sync_copy(zeros_hbm.at[pl.ds(NS * RPS, RPS_TAIL)],
                        acc.at[pl.ds(NS * RPS, RPS_TAIL)])
    plsc.subcore_barrier()


def _acc_writeout(acc, out_hbm, c, s):
    plsc.subcore_barrier()
    pltpu.sync_copy(acc.at[pl.ds(s * RPW, RPW)],
                    out_hbm.at[c, pl.ds(s * RPW, RPW)])

    @pl.when(s == NS - 1)
    def _():
        pltpu.sync_copy(acc.at[pl.ds(NS * RPW, RPW_TAIL)],
                        out_hbm.at[c, pl.ds(NS * RPW, RPW_TAIL)])


@functools.cache
def _edge_agg():
    mesh = plsc.VectorSubcoreMesh(core_axis_name="c", subcore_axis_name="s")

    @functools.partial(
        pl.kernel,
        out_type=jax.ShapeDtypeStruct((NC, HALF, H), jnp.float32),
        mesh=mesh,
        scratch_types=[
            pltpu.VMEM((CH, K), jnp.int32),
            pltpu.VMEM((CH, K), jnp.int32),
            pltpu.VMEM((K, H), jnp.float32),
            pltpu.VMEM((K, H), jnp.float32),
            pltpu.VMEM_SHARED((ACC_R, H), jnp.float32),
            pltpu.SemaphoreType.DMA,
            pltpu.SemaphoreType.DMA,
        ],
    )
    def k(src_hbm, dstloc_hbm, y_hbm, zeros_hbm, out_hbm,
          srcv, dstv, buf0, buf1, acc, sem0, sem1):
        c = lax.axis_index("c")
        s = lax.axis_index("s")
        pltpu.sync_copy(src_hbm.at[s], srcv)
        pltpu.sync_copy(dstloc_hbm.at[c, s], dstv)
        _acc_setup(zeros_hbm, acc, s)

        def wait(buf, sem):
            pltpu.make_async_copy(y_hbm.at[srcv.at[0]], buf, sem).wait()

        pltpu.async_copy(y_hbm.at[srcv.at[0]], buf0, sem0)

        def pair(i, carry):
            j = 2 * i
            wait(buf0, sem0)
            pltpu.async_copy(y_hbm.at[srcv.at[j + 1]], buf1, sem1)
            pltpu.sync_copy(buf0, acc.at[dstv.at[j]], add=True)
            wait(buf1, sem1)
            pltpu.async_copy(y_hbm.at[srcv.at[j + 2]], buf0, sem0)
            pltpu.sync_copy(buf1, acc.at[dstv.at[j + 1]], add=True)
            return carry

        lax.fori_loop(0, CH // 2 - 1, pair, 0)
        wait(buf0, sem0)
        pltpu.async_copy(y_hbm.at[srcv.at[CH - 1]], buf1, sem1)
        pltpu.sync_copy(buf0, acc.at[dstv.at[CH - 2]], add=True)
        wait(buf1, sem1)
        pltpu.sync_copy(buf1, acc.at[dstv.at[CH - 1]], add=True)

        _acc_writeout(acc, out_hbm, c, s)

    return k


@functools.cache
def _deg_kernel():
    mesh = plsc.VectorSubcoreMesh(core_axis_name="c", subcore_axis_name="s")

    @functools.partial(
        pl.kernel,
        out_type=jax.ShapeDtypeStruct((NC, HALF, H), jnp.float32),
        mesh=mesh,
        scratch_types=[
            pltpu.VMEM((CH, K), jnp.int32),
            pltpu.VMEM((K, H), jnp.float32),
            pltpu.VMEM_SHARED((ACC_R, H), jnp.float32),
        ],
    )
    def k(dstloc_hbm, ones_hbm, zeros_hbm, out_hbm, dstv, onesv, acc):
        c = lax.axis_index("c")
        s = lax.axis_index("s")
        pltpu.sync_copy(dstloc_hbm.at[c, s], dstv)
        pltpu.sync_copy(ones_hbm, onesv)
        _acc_setup(zeros_hbm, acc, s)

        def body(j, carry):
            pltpu.sync_copy(onesv, acc.at[dstv.at[j]], add=True)
            return carry

        lax.fori_loop(0, CH, body, 0)

        _acc_writeout(acc, out_hbm, c, s)

    return k


def _dstloc_body(d_ref, o_ref):
    c = pl.program_id(0)
    d = d_ref[...]
    dl = d - c * HALF
    keep = (dl >= 0) & (dl < HALF)
    o_ref[0] = jnp.where(keep, dl, HALF + (d & (TRASH - 1)))


@functools.cache
def _dstloc_call():
    return pl.pallas_call(
        _dstloc_body,
        grid=(NC,),
        in_specs=[pl.BlockSpec((E // 128, 128), lambda c: (0, 0))],
        out_specs=pl.BlockSpec((1, E // 128, 128), lambda c: (c, 0, 0)),
        out_shape=jax.ShapeDtypeStruct((NC, E // 128, 128), jnp.int32),
    )


def _sigm(v):
    return 1.0 / (1.0 + jnp.exp(-v))


def _lstm_body(x_ref, wi0_ref, wh0_ref, b0a_ref, b0b_ref,
               wi1_ref, wh1_ref, b1a_ref, b1b_ref, o_ref):
    bn = x_ref.shape[0]
    b0 = b0a_ref[...] + b0b_ref[...]
    b1 = b1a_ref[...] + b1b_ref[...]
    h0 = jnp.zeros((bn, H), jnp.float32)
    c0 = jnp.zeros((bn, H), jnp.float32)
    h1 = jnp.zeros((bn, H), jnp.float32)
    c1 = jnp.zeros((bn, H), jnp.float32)
    for t in range(T):
        xt = x_ref[:, 2 * t:2 * t + 2]
        g = (jnp.dot(xt, wi0_ref[...], preferred_element_type=jnp.float32)
             + jnp.dot(h0, wh0_ref[...], preferred_element_type=jnp.float32)
             + b0)
        i = _sigm(g[:, 0 * H:1 * H])
        f = _sigm(g[:, 1 * H:2 * H])
        gg = jnp.tanh(g[:, 2 * H:3 * H])
        o = _sigm(g[:, 3 * H:4 * H])
        c0 = f * c0 + i * gg
        h0 = o * jnp.tanh(c0)
        g = (jnp.dot(h0, wi1_ref[...], preferred_element_type=jnp.float32)
             + jnp.dot(h1, wh1_ref[...], preferred_element_type=jnp.float32)
             + b1)
        i = _sigm(g[:, 0 * H:1 * H])
        f = _sigm(g[:, 1 * H:2 * H])
        gg = jnp.tanh(g[:, 2 * H:3 * H])
        o = _sigm(g[:, 3 * H:4 * H])
        c1 = f * c1 + i * gg
        h1 = o * jnp.tanh(c1)
    o_ref[...] = h1


BN_LSTM = 1000
BN_MM = 2000


@functools.cache
def _lstm_call():
    full = lambda shape: pl.BlockSpec(shape, lambda i: (0,) * len(shape))
    return pl.pallas_call(
        _lstm_body,
        grid=(N // BN_LSTM,),
        in_specs=[
            pl.BlockSpec((BN_LSTM, T * IN_DIM), lambda i: (i, 0)),
            full((IN_DIM, 4 * H)), full((H, 4 * H)),
            full((1, 4 * H)), full((1, 4 * H)),
            full((H, 4 * H)), full((H, 4 * H)),
            full((1, 4 * H)), full((1, 4 * H)),
        ],
        out_specs=pl.BlockSpec((BN_LSTM, H), lambda i: (i, 0)),
        out_shape=jax.ShapeDtypeStruct((N, H), jnp.float32),
    )


def _mm_first_body(h_ref, w_ref, deg_ref, y_ref, dinv_ref):
    dv = lax.rsqrt(deg_ref[:, 0:1] + 1.0)
    dinv_ref[...] = jnp.broadcast_to(dv, dinv_ref.shape)
    y_ref[...] = (jnp.dot(h_ref[...], w_ref[...],
                          preferred_element_type=jnp.float32) * dv)


@functools.cache
def _mm_first_call():
    return pl.pallas_call(
        _mm_first_body,
        grid=(N // BN_MM,),
        in_specs=[
            pl.BlockSpec((BN_MM, H), lambda i: (i, 0)),
            pl.BlockSpec((H, H), lambda i: (0, 0)),
            pl.BlockSpec((BN_MM, H), lambda i: (i, 0)),
        ],
        out_specs=[
            pl.BlockSpec((BN_MM, H), lambda i: (i, 0)),
            pl.BlockSpec((BN_MM, 16), lambda i: (i, 0)),
        ],
        out_shape=[
            jax.ShapeDtypeStruct((N, H), jnp.float32),
            jax.ShapeDtypeStruct((N, 16), jnp.float32),
        ],
    )


def _mm_mid_body(agg_ref, y_ref, dinv_ref, b_ref, w_ref, o_ref):
    dv = dinv_ref[:, 0:1]
    pre = (agg_ref[...] + y_ref[...]) * dv + b_ref[...]
    hrelu = jnp.maximum(pre, 0.0)
    o_ref[...] = (jnp.dot(hrelu, w_ref[...],
                          preferred_element_type=jnp.float32) * dv)


@functools.cache
def _mm_mid_call():
    return pl.pallas_call(
        _mm_mid_body,
        grid=(N // BN_MM,),
        in_specs=[
            pl.BlockSpec((BN_MM, H), lambda i: (i, 0)),
            pl.BlockSpec((BN_MM, H), lambda i: (i, 0)),
            pl.BlockSpec((BN_MM, 16), lambda i: (i, 0)),
            pl.BlockSpec((1, H), lambda i: (0, 0)),
            pl.BlockSpec((H, H), lambda i: (0, 0)),
        ],
        out_specs=pl.BlockSpec((BN_MM, H), lambda i: (i, 0)),
        out_shape=jax.ShapeDtypeStruct((N, H), jnp.float32),
    )


def _mm_scale_body(agg_ref, y_ref, dinv_ref, b_ref, o_ref):
    dv = dinv_ref[:, 0:1]
    pre = (agg_ref[...] + y_ref[...]) * dv + b_ref[...]
    o_ref[...] = jnp.maximum(pre, 0.0) * dv


@functools.cache
def _mm_scale_call():
    return pl.pallas_call(
        _mm_scale_body,
        grid=(N // BN_MM,),
        in_specs=[
            pl.BlockSpec((BN_MM, H), lambda i: (i, 0)),
            pl.BlockSpec((BN_MM, H), lambda i: (i, 0)),
            pl.BlockSpec((BN_MM, 16), lambda i: (i, 0)),
            pl.BlockSpec((1, H), lambda i: (0, 0)),
        ],
        out_specs=pl.BlockSpec((BN_MM, H), lambda i: (i, 0)),
        out_shape=jax.ShapeDtypeStruct((N, H), jnp.float32),
    )


def _final_body(agg_ref, u_ref, dinv_ref, w_ref, b_ref, o_ref):
    du = (agg_ref[...] + u_ref[...]) * dinv_ref[:, 0:1]
    o_ref[...] = (jnp.dot(du, w_ref[...],
                          preferred_element_type=jnp.float32) + b_ref[...])


@functools.cache
def _final_call():
    return pl.pallas_call(
        _final_body,
        grid=(N // BN_MM,),
        in_specs=[
            pl.BlockSpec((BN_MM, H), lambda i: (i, 0)),
            pl.BlockSpec((BN_MM, H), lambda i: (i, 0)),
            pl.BlockSpec((BN_MM, 16), lambda i: (i, 0)),
            pl.BlockSpec((H, 16), lambda i: (0, 0)),
            pl.BlockSpec((1, 16), lambda i: (0, 0)),
        ],
        out_specs=pl.BlockSpec((BN_MM, 16), lambda i: (i, 0)),
        out_shape=jax.ShapeDtypeStruct((N, 16), jnp.float32),
    )


def kernel(x, edge_index, W_ih0, W_hh0, b_ih0, b_hh0,
           W_ih1, W_hh1, b_ih1, b_hh1, Wg1, bg1, Wg2, bg2, Wg3, bg3):
    x24 = x.reshape(N, T * IN_DIM)
    src2 = edge_index[0].reshape(NS, CH, K)
    dstloc = _dstloc_call()(edge_index[1].reshape(E // 128, 128))
    dstloc4 = dstloc.reshape(NC, NS, CH, K)
    onesK = jnp.ones((K, H), jnp.float32)
    zacc = jnp.zeros((ACC_R, H), jnp.float32)

    h = _lstm_call()(
        x24, W_ih0.T, W_hh0.T, b_ih0.reshape(1, -1), b_hh0.reshape(1, -1),
        W_ih1.T, W_hh1.T, b_ih1.reshape(1, -1), b_hh1.reshape(1, -1))

    deg = _deg_kernel()(dstloc4, onesK, zacc).reshape(N, H)
    y1, dinv16 = _mm_first_call()(h, Wg1, deg)

    agg1 = _edge_agg()(src2, dstloc4, y1, zacc).reshape(N, H)
    y2 = _mm_mid_call()(agg1, y1, dinv16, bg1.reshape(1, H), Wg2)

    agg2 = _edge_agg()(src2, dstloc4, y2, zacc).reshape(N, H)
    u3 = _mm_scale_call()(agg2, y2, dinv16, bg2.reshape(1, H))

    agg3 = _edge_agg()(src2, dstloc4, u3, zacc).reshape(N, H)
    wg3p = jnp.zeros((H, 16), jnp.float32).at[:, :OUT_DIM].set(Wg3)
    bg3p = jnp.zeros((1, 16), jnp.float32).at[:, :OUT_DIM].set(bg3.reshape(1, -1))
    out16 = _final_call()(agg3, u3, dinv16, wg3p, bg3p)
    return out16[:, :OUT_DIM]


# speedup vs baseline: 13.2489x; 1.2497x over previous
import functools

import jax
import jax.numpy as jnp
from jax import lax
from jax.experimental import pallas as pl
from jax.experimental.pallas import tpu as pltpu
from jax.experimental.pallas import tpu_sc as plsc

N = 10000
T = 12
IN_DIM = 2
H = 128
OUT_DIM = 12
E = 320000

NC = 2
NS = 16
NW = NC * NS
HALF = N // NC
TRASH = 96
ACC_R = HALF + TRASH
EPS = E // NS
K = 128
CHP = EPS // K + 2
CHPK = CHP * K
RPS = 320
RPS_LAST = ACC_R - (NS - 1) * RPS
RPW = 312
RPW_TAIL = HALF - NS * RPW

_SC_PARAMS = pltpu.CompilerParams(needs_layout_passes=False)


def _acc_setup(zeros_hbm, acc, s):
    @pl.when(s < NS - 1)
    def _():
        pltpu.sync_copy(zeros_hbm.at[pl.ds(s * RPS, RPS)],
                        acc.at[pl.ds(s * RPS, RPS)])

    @pl.when(s == NS - 1)
    def _():
        pltpu.sync_copy(zeros_hbm.at[pl.ds((NS - 1) * RPS, RPS_LAST)],
                        acc.at[pl.ds((NS - 1) * RPS, RPS_LAST)])
    plsc.subcore_barrier()


def _acc_writeout(acc, out_hbm, c, s):
    plsc.subcore_barrier()
    pltpu.sync_copy(acc.at[pl.ds(s * RPW, RPW)],
                    out_hbm.at[c, pl.ds(s * RPW, RPW)])

    @pl.when(s == NS - 1)
    def _():
        pltpu.sync_copy(acc.at[pl.ds(NS * RPW, RPW_TAIL)],
                        out_hbm.at[c, pl.ds(NS * RPW, RPW_TAIL)])


@functools.cache
def _compact_kernel():
    mesh = plsc.VectorSubcoreMesh(core_axis_name="c", subcore_axis_name="s")

    @functools.partial(
        pl.kernel,
        out_type=[
            jax.ShapeDtypeStruct((NW, CHP, K), jnp.int32),
            jax.ShapeDtypeStruct((NW, CHP + 1, K), jnp.int32),
        ],
        mesh=mesh,
        compiler_params=_SC_PARAMS,
        scratch_types=[
            pltpu.VMEM((EPS,), jnp.int32),
            pltpu.VMEM((EPS,), jnp.int32),
            pltpu.VMEM((CHPK + 16,), jnp.int32),
            pltpu.VMEM((CHPK + 16,), jnp.int32),
            pltpu.VMEM((CHP, K), jnp.int32),
            pltpu.VMEM((CHP + 1, K), jnp.int32),
        ],
    )
    def k(src_hbm, dst_hbm, csrc_hbm, cdst_hbm,
          sflat, dflat, csflat, cdflat, cs2d, cd2d):
        c = lax.axis_index("c")
        s = lax.axis_index("s")
        w = c * NS + s
        pltpu.sync_copy(src_hbm.at[s], sflat)
        pltpu.sync_copy(dst_hbm.at[s], dflat)

        iota = lax.iota(jnp.int32, 16)

        def prefix_sum(v):
            for step in (1, 2, 4, 8):
                idx = jnp.maximum(iota - step, 0)
                g = lax.gather(
                    v, idx[:, None],
                    lax.GatherDimensionNumbers(
                        offset_dims=(), collapsed_slice_dims=(0,),
                        start_index_map=(0,)),
                    (1,), mode=lax.GatherScatterMode.PROMISE_IN_BOUNDS)
                v = v + jnp.where(iota >= step, g, 0)
            return v

        def body(i, off):
            sv = sflat[pl.ds(i * 16, 16)]
            dv = dflat[pl.ds(i * 16, 16)]
            dl = dv - c * HALF
            m = (dl >= 0) & (dl < HALF)
            pre = prefix_sum(jnp.where(m, 1, 0))
            pos = jnp.where(m, off + pre - 1, CHPK + iota)
            plsc.store_scatter(csflat, [pos], sv)
            plsc.store_scatter(cdflat, [pos], dl)
            return off + pre[15]

        off = lax.fori_loop(0, EPS // 16, body, 0)

        for g in range(K // 16):
            csflat[pl.ds(off + 16 * g, 16)] = jnp.zeros((16,), jnp.int32)
            cdflat[pl.ds(off + 16 * g, 16)] = HALF + iota + 16 * g
        ncho = (off + K - 1) // K
        cd2d[CHP, pl.ds(0, 16)] = jnp.broadcast_to(ncho, (16,)).astype(jnp.int32)

        def rc(r, carry):
            for q in range(K // 16):
                cs2d[r, pl.ds(16 * q, 16)] = csflat[pl.ds(r * K + 16 * q, 16)]
                cd2d[r, pl.ds(16 * q, 16)] = cdflat[pl.ds(r * K + 16 * q, 16)]
            return carry

        lax.fori_loop(0, CHP, rc, 0)

        pltpu.sync_copy(cs2d, csrc_hbm.at[w])
        pltpu.sync_copy(cd2d, cdst_hbm.at[w])

    return k


@functools.cache
def _edge_agg():
    mesh = plsc.VectorSubcoreMesh(core_axis_name="c", subcore_axis_name="s")

    @functools.partial(
        pl.kernel,
        out_type=jax.ShapeDtypeStruct((NC, HALF, H), jnp.float32),
        mesh=mesh,
        compiler_params=_SC_PARAMS,
        scratch_types=[
            pltpu.VMEM((CHP, K), jnp.int32),
            pltpu.VMEM((CHP + 1, K), jnp.int32),
            pltpu.VMEM((K, H), jnp.float32),
            pltpu.VMEM((K, H), jnp.float32),
            pltpu.VMEM_SHARED((ACC_R, H), jnp.float32),
            pltpu.SemaphoreType.DMA,
            pltpu.SemaphoreType.DMA,
        ],
    )
    def k(csrc_hbm, cdst_hbm, y_hbm, zeros_hbm, out_hbm,
          srcv, dstv, buf0, buf1, acc, sem0, sem1):
        c = lax.axis_index("c")
        s = lax.axis_index("s")
        w = c * NS + s
        pltpu.sync_copy(csrc_hbm.at[w], srcv)
        pltpu.sync_copy(cdst_hbm.at[w], dstv)
        ncho = dstv[CHP, pl.ds(0, 16)][0]
        _acc_setup(zeros_hbm, acc, s)

        def wait(buf, sem):
            pltpu.make_async_copy(y_hbm.at[srcv.at[0]], buf, sem).wait()

        @pl.when(ncho > 0)
        def _():
            pltpu.async_copy(y_hbm.at[srcv.at[0]], buf0, sem0)

        def body(j, carry):
            @pl.when(lax.rem(j, 2) == 0)
            def _():
                wait(buf0, sem0)

                @pl.when(j + 1 < ncho)
                def _():
                    pltpu.async_copy(y_hbm.at[srcv.at[j + 1]], buf1, sem1)
                pltpu.sync_copy(buf0, acc.at[dstv.at[j]], add=True)

            @pl.when(lax.rem(j, 2) == 1)
            def _():
                wait(buf1, sem1)

                @pl.when(j + 1 < ncho)
                def _():
                    pltpu.async_copy(y_hbm.at[srcv.at[j + 1]], buf0, sem0)
                pltpu.sync_copy(buf1, acc.at[dstv.at[j]], add=True)
            return carry

        lax.fori_loop(0, ncho, body, 0)

        _acc_writeout(acc, out_hbm, c, s)

    return k


@functools.cache
def _deg_kernel():
    mesh = plsc.VectorSubcoreMesh(core_axis_name="c", subcore_axis_name="s")

    @functools.partial(
        pl.kernel,
        out_type=jax.ShapeDtypeStruct((NC, HALF, H), jnp.float32),
        mesh=mesh,
        compiler_params=_SC_PARAMS,
        scratch_types=[
            pltpu.VMEM((CHP + 1, K), jnp.int32),
            pltpu.VMEM((K, H), jnp.float32),
            pltpu.VMEM_SHARED((ACC_R, H), jnp.float32),
        ],
    )
    def k(cdst_hbm, ones_hbm, zeros_hbm, out_hbm,
          dstv, onesv, acc):
        c = lax.axis_index("c")
        s = lax.axis_index("s")
        w = c * NS + s
        pltpu.sync_copy(cdst_hbm.at[w], dstv)
        pltpu.sync_copy(ones_hbm, onesv)
        ncho = dstv[CHP, pl.ds(0, 16)][0]
        _acc_setup(zeros_hbm, acc, s)

        def body(j, carry):
            pltpu.sync_copy(onesv, acc.at[dstv.at[j]], add=True)
            return carry

        lax.fori_loop(0, ncho, body, 0)

        _acc_writeout(acc, out_hbm, c, s)

    return k


def _sigm(v):
    return 1.0 / (1.0 + jnp.exp(-v))


def _lstm_body(x_ref, wi0_ref, wh0_ref, b0a_ref, b0b_ref,
               wi1_ref, wh1_ref, b1a_ref, b1b_ref, o_ref):
    bn = x_ref.shape[0]
    b0 = b0a_ref[...] + b0b_ref[...]
    b1 = b1a_ref[...] + b1b_ref[...]
    h0 = jnp.zeros((bn, H), jnp.float32)
    c0 = jnp.zeros((bn, H), jnp.float32)
    h1 = jnp.zeros((bn, H), jnp.float32)
    c1 = jnp.zeros((bn, H), jnp.float32)
    for t in range(T):
        xt = x_ref[:, 2 * t:2 * t + 2]
        g = (jnp.dot(xt, wi0_ref[...], preferred_element_type=jnp.float32)
             + jnp.dot(h0, wh0_ref[...], preferred_element_type=jnp.float32)
             + b0)
        i = _sigm(g[:, 0 * H:1 * H])
        f = _sigm(g[:, 1 * H:2 * H])
        gg = jnp.tanh(g[:, 2 * H:3 * H])
        o = _sigm(g[:, 3 * H:4 * H])
        c0 = f * c0 + i * gg
        h0 = o * jnp.tanh(c0)
        g = (jnp.dot(h0, wi1_ref[...], preferred_element_type=jnp.float32)
             + jnp.dot(h1, wh1_ref[...], preferred_element_type=jnp.float32)
             + b1)
        i = _sigm(g[:, 0 * H:1 * H])
        f = _sigm(g[:, 1 * H:2 * H])
        gg = jnp.tanh(g[:, 2 * H:3 * H])
        o = _sigm(g[:, 3 * H:4 * H])
        c1 = f * c1 + i * gg
        h1 = o * jnp.tanh(c1)
    o_ref[...] = h1


BN_LSTM = 1000
BN_MM = 2000


@functools.cache
def _lstm_call():
    full = lambda shape: pl.BlockSpec(shape, lambda i: (0,) * len(shape))
    return pl.pallas_call(
        _lstm_body,
        grid=(N // BN_LSTM,),
        in_specs=[
            pl.BlockSpec((BN_LSTM, T * IN_DIM), lambda i: (i, 0)),
            full((IN_DIM, 4 * H)), full((H, 4 * H)),
            full((1, 4 * H)), full((1, 4 * H)),
            full((H, 4 * H)), full((H, 4 * H)),
            full((1, 4 * H)), full((1, 4 * H)),
        ],
        out_specs=pl.BlockSpec((BN_LSTM, H), lambda i: (i, 0)),
        out_shape=jax.ShapeDtypeStruct((N, H), jnp.float32),
    )


def _mm_first_body(h_ref, w_ref, deg_ref, y_ref, dinv_ref):
    dv = lax.rsqrt(deg_ref[:, 0:1] + 1.0)
    dinv_ref[...] = jnp.broadcast_to(dv, dinv_ref.shape)
    y_ref[...] = (jnp.dot(h_ref[...], w_ref[...],
                          preferred_element_type=jnp.float32) * dv)


@functools.cache
def _mm_first_call():
    return pl.pallas_call(
        _mm_first_body,
        grid=(N // BN_MM,),
        in_specs=[
            pl.BlockSpec((BN_MM, H), lambda i: (i, 0)),
            pl.BlockSpec((H, H), lambda i: (0, 0)),
            pl.BlockSpec((BN_MM, H), lambda i: (i, 0)),
        ],
        out_specs=[
            pl.BlockSpec((BN_MM, H), lambda i: (i, 0)),
            pl.BlockSpec((BN_MM, 16), lambda i: (i, 0)),
        ],
        out_shape=[
            jax.ShapeDtypeStruct((N, H), jnp.float32),
            jax.ShapeDtypeStruct((N, 16), jnp.float32),
        ],
    )


def _mm_mid_body(agg_ref, y_ref, dinv_ref, b_ref, w_ref, o_ref):
    dv = dinv_ref[:, 0:1]
    pre = (agg_ref[...] + y_ref[...]) * dv + b_ref[...]
    hrelu = jnp.maximum(pre, 0.0)
    o_ref[...] = (jnp.dot(hrelu, w_ref[...],
                          preferred_element_type=jnp.float32) * dv)


@functools.cache
def _mm_mid_call():
    return pl.pallas_call(
        _mm_mid_body,
        grid=(N // BN_MM,),
        in_specs=[
            pl.BlockSpec((BN_MM, H), lambda i: (i, 0)),
            pl.BlockSpec((BN_MM, H), lambda i: (i, 0)),
            pl.BlockSpec((BN_MM, 16), lambda i: (i, 0)),
            pl.BlockSpec((1, H), lambda i: (0, 0)),
            pl.BlockSpec((H, H), lambda i: (0, 0)),
        ],
        out_specs=pl.BlockSpec((BN_MM, H), lambda i: (i, 0)),
        out_shape=jax.ShapeDtypeStruct((N, H), jnp.float32),
    )


def _mm_scale_body(agg_ref, y_ref, dinv_ref, b_ref, o_ref):
    dv = dinv_ref[:, 0:1]
    pre = (agg_ref[...] + y_ref[...]) * dv + b_ref[...]
    o_ref[...] = jnp.maximum(pre, 0.0) * dv


@functools.cache
def _mm_scale_call():
    return pl.pallas_call(
        _mm_scale_body,
        grid=(N // BN_MM,),
        in_specs=[
            pl.BlockSpec((BN_MM, H), lambda i: (i, 0)),
            pl.BlockSpec((BN_MM, H), lambda i: (i, 0)),
            pl.BlockSpec((BN_MM, 16), lambda i: (i, 0)),
            pl.BlockSpec((1, H), lambda i: (0, 0)),
        ],
        out_specs=pl.BlockSpec((BN_MM, H), lambda i: (i, 0)),
        out_shape=jax.ShapeDtypeStruct((N, H), jnp.float32),
    )


def _final_body(agg_ref, u_ref, dinv_ref, w_ref, b_ref, o_ref):
    du = (agg_ref[...] + u_ref[...]) * dinv_ref[:, 0:1]
    o_ref[...] = (jnp.dot(du, w_ref[...],
                          preferred_element_type=jnp.float32) + b_ref[...])


@functools.cache
def _final_call():
    return pl.pallas_call(
        _final_body,
        grid=(N // BN_MM,),
        in_specs=[
            pl.BlockSpec((BN_MM, H), lambda i: (i, 0)),
            pl.BlockSpec((BN_MM, H), lambda i: (i, 0)),
            pl.BlockSpec((BN_MM, 16), lambda i: (i, 0)),
            pl.BlockSpec((H, 16), lambda i: (0, 0)),
            pl.BlockSpec((1, 16), lambda i: (0, 0)),
        ],
        out_specs=pl.BlockSpec((BN_MM, 16), lambda i: (i, 0)),
        out_shape=jax.ShapeDtypeStruct((N, 16), jnp.float32),
    )


def kernel(x, edge_index, W_ih0, W_hh0, b_ih0, b_hh0,
           W_ih1, W_hh1, b_ih1, b_hh1, Wg1, bg1, Wg2, bg2, Wg3, bg3):
    x24 = x.reshape(N, T * IN_DIM)
    src2 = edge_index[0].reshape(NS, EPS)
    dst2 = edge_index[1].reshape(NS, EPS)
    onesK = jnp.ones((K, H), jnp.float32)
    zacc = jnp.zeros((ACC_R, H), jnp.float32)

    csrc, cdst = _compact_kernel()(src2, dst2)

    h = _lstm_call()(
        x24, W_ih0.T, W_hh0.T, b_ih0.reshape(1, -1), b_hh0.reshape(1, -1),
        W_ih1.T, W_hh1.T, b_ih1.reshape(1, -1), b_hh1.reshape(1, -1))

    deg = _deg_kernel()(cdst, onesK, zacc).reshape(N, H)
    y1, dinv16 = _mm_first_call()(h, Wg1, deg)

    agg1 = _edge_agg()(csrc, cdst, y1, zacc).reshape(N, H)
    y2 = _mm_mid_call()(agg1, y1, dinv16, bg1.reshape(1, H), Wg2)

    agg2 = _edge_agg()(csrc, cdst, y2, zacc).reshape(N, H)
    u3 = _mm_scale_call()(agg2, y2, dinv16, bg2.reshape(1, H))

    agg3 = _edge_agg()(csrc, cdst, u3, zacc).reshape(N, H)
    wg3p = jnp.zeros((H, 16), jnp.float32).at[:, :OUT_DIM].set(Wg3)
    bg3p = jnp.zeros((1, 16), jnp.float32).at[:, :OUT_DIM].set(bg3.reshape(1, -1))
    out16 = _final_call()(agg3, u3, dinv16, wg3p, bg3p)
    return out16[:, :OUT_DIM]


# speedup vs baseline: 13.4583x; 1.0158x over previous
import functools

import jax
import jax.numpy as jnp
from jax import lax
from jax.experimental import pallas as pl
from jax.experimental.pallas import tpu as pltpu
from jax.experimental.pallas import tpu_sc as plsc

N = 10000
T = 12
IN_DIM = 2
H = 128
OUT_DIM = 12
E = 320000

NC = 2
NS = 16
NW = NC * NS
HALF = N // NC
TRASH = 96
ACC_R = HALF + TRASH
EPS = E // NS
K = 128
CHP = EPS // K + 2
CHPK = CHP * K
RPS = 320
RPS_LAST = ACC_R - (NS - 1) * RPS
RPW = 312
RPW_TAIL = HALF - NS * RPW

_SC_PARAMS = pltpu.CompilerParams(needs_layout_passes=False)


def _acc_setup(zeros_hbm, acc, s):
    @pl.when(s < NS - 1)
    def _():
        pltpu.sync_copy(zeros_hbm.at[pl.ds(s * RPS, RPS)],
                        acc.at[pl.ds(s * RPS, RPS)])

    @pl.when(s == NS - 1)
    def _():
        pltpu.sync_copy(zeros_hbm.at[pl.ds((NS - 1) * RPS, RPS_LAST)],
                        acc.at[pl.ds((NS - 1) * RPS, RPS_LAST)])
    plsc.subcore_barrier()


def _acc_writeout(acc, out_hbm, c, s):
    plsc.subcore_barrier()
    pltpu.sync_copy(acc.at[pl.ds(s * RPW, RPW)],
                    out_hbm.at[c, pl.ds(s * RPW, RPW)])

    @pl.when(s == NS - 1)
    def _():
        pltpu.sync_copy(acc.at[pl.ds(NS * RPW, RPW_TAIL)],
                        out_hbm.at[c, pl.ds(NS * RPW, RPW_TAIL)])


@functools.cache
def _compact_kernel():
    mesh = plsc.VectorSubcoreMesh(core_axis_name="c", subcore_axis_name="s")

    @functools.partial(
        pl.kernel,
        out_type=[
            jax.ShapeDtypeStruct((NW, CHP, K), jnp.int32),
            jax.ShapeDtypeStruct((NW, CHP + 1, K), jnp.int32),
        ],
        mesh=mesh,
        compiler_params=_SC_PARAMS,
        scratch_types=[
            pltpu.VMEM((EPS,), jnp.int32),
            pltpu.VMEM((EPS,), jnp.int32),
            pltpu.VMEM((CHPK + 16,), jnp.int32),
            pltpu.VMEM((CHPK + 16,), jnp.int32),
            pltpu.VMEM((CHP, K), jnp.int32),
            pltpu.VMEM((CHP + 1, K), jnp.int32),
        ],
    )
    def k(src_hbm, dst_hbm, csrc_hbm, cdst_hbm,
          sflat, dflat, csflat, cdflat, cs2d, cd2d):
        c = lax.axis_index("c")
        s = lax.axis_index("s")
        w = c * NS + s
        pltpu.sync_copy(src_hbm.at[s], sflat)
        pltpu.sync_copy(dst_hbm.at[s], dflat)

        iota = lax.iota(jnp.int32, 16)

        def prefix_sum(v):
            for step in (1, 2, 4, 8):
                idx = jnp.maximum(iota - step, 0)
                g = lax.gather(
                    v, idx[:, None],
                    lax.GatherDimensionNumbers(
                        offset_dims=(), collapsed_slice_dims=(0,),
                        start_index_map=(0,)),
                    (1,), mode=lax.GatherScatterMode.PROMISE_IN_BOUNDS)
                v = v + jnp.where(iota >= step, g, 0)
            return v

        def body(i, off):
            sv = sflat[pl.ds(i * 16, 16)]
            dv = dflat[pl.ds(i * 16, 16)]
            dl = dv - c * HALF
            m = (dl >= 0) & (dl < HALF)
            pre = prefix_sum(jnp.where(m, 1, 0))
            pos = jnp.where(m, off + pre - 1, CHPK + iota)
            plsc.store_scatter(csflat, [pos], sv)
            plsc.store_scatter(cdflat, [pos], dl)
            return off + pre[15]

        off = lax.fori_loop(0, EPS // 16, body, 0)

        for g in range(K // 16):
            csflat[pl.ds(off + 16 * g, 16)] = jnp.zeros((16,), jnp.int32)
            cdflat[pl.ds(off + 16 * g, 16)] = HALF + iota + 16 * g
        ncho = (off + K - 1) // K
        cd2d[CHP, pl.ds(0, 16)] = jnp.broadcast_to(ncho, (16,)).astype(jnp.int32)

        def rc(r, carry):
            for q in range(K // 16):
                cs2d[r, pl.ds(16 * q, 16)] = csflat[pl.ds(r * K + 16 * q, 16)]
                cd2d[r, pl.ds(16 * q, 16)] = cdflat[pl.ds(r * K + 16 * q, 16)]
            return carry

        lax.fori_loop(0, CHP, rc, 0)

        pltpu.sync_copy(cs2d, csrc_hbm.at[w])
        pltpu.sync_copy(cd2d, cdst_hbm.at[w])

    return k


@functools.cache
def _edge_agg():
    mesh = plsc.VectorSubcoreMesh(core_axis_name="c", subcore_axis_name="s")

    @functools.partial(
        pl.kernel,
        out_type=jax.ShapeDtypeStruct((NC, HALF, H), jnp.float32),
        mesh=mesh,
        compiler_params=_SC_PARAMS,
        scratch_types=[
            pltpu.VMEM((CHP, K), jnp.int32),
            pltpu.VMEM((CHP + 1, K), jnp.int32),
            pltpu.VMEM((K, H), jnp.float32),
            pltpu.VMEM((K, H), jnp.float32),
            pltpu.VMEM_SHARED((ACC_R, H), jnp.float32),
            pltpu.SemaphoreType.DMA,
            pltpu.SemaphoreType.DMA,
        ],
    )
    def k(csrc_hbm, cdst_hbm, y_hbm, zeros_hbm, out_hbm,
          srcv, dstv, buf0, buf1, acc, sem0, sem1):
        c = lax.axis_index("c")
        s = lax.axis_index("s")
        w = c * NS + s
        pltpu.sync_copy(csrc_hbm.at[w], srcv)
        pltpu.sync_copy(cdst_hbm.at[w], dstv)
        ncho = dstv[CHP, pl.ds(0, 16)][0]
        _acc_setup(zeros_hbm, acc, s)

        def wait(buf, sem):
            pltpu.make_async_copy(y_hbm.at[srcv.at[0]], buf, sem).wait()

        @pl.when(ncho > 0)
        def _():
            pltpu.async_copy(y_hbm.at[srcv.at[0]], buf0, sem0)

        def body(j, carry):
            @pl.when(lax.rem(j, 2) == 0)
            def _():
                wait(buf0, sem0)

                @pl.when(j + 1 < ncho)
                def _():
                    pltpu.async_copy(y_hbm.at[srcv.at[j + 1]], buf1, sem1)
                pltpu.sync_copy(buf0, acc.at[dstv.at[j]], add=True)

            @pl.when(lax.rem(j, 2) == 1)
            def _():
                wait(buf1, sem1)

                @pl.when(j + 1 < ncho)
                def _():
                    pltpu.async_copy(y_hbm.at[srcv.at[j + 1]], buf0, sem0)
                pltpu.sync_copy(buf1, acc.at[dstv.at[j]], add=True)
            return carry

        lax.fori_loop(0, ncho, body, 0)

        _acc_writeout(acc, out_hbm, c, s)

    return k


@functools.cache
def _deg_kernel():
    mesh = plsc.VectorSubcoreMesh(core_axis_name="c", subcore_axis_name="s")

    @functools.partial(
        pl.kernel,
        out_type=jax.ShapeDtypeStruct((NC, HALF, H), jnp.float32),
        mesh=mesh,
        compiler_params=_SC_PARAMS,
        scratch_types=[
            pltpu.VMEM((CHP + 1, K), jnp.int32),
            pltpu.VMEM((K, H), jnp.float32),
            pltpu.VMEM_SHARED((ACC_R, H), jnp.float32),
        ],
    )
    def k(cdst_hbm, ones_hbm, zeros_hbm, out_hbm,
          dstv, onesv, acc):
        c = lax.axis_index("c")
        s = lax.axis_index("s")
        w = c * NS + s
        pltpu.sync_copy(cdst_hbm.at[w], dstv)
        pltpu.sync_copy(ones_hbm, onesv)
        ncho = dstv[CHP, pl.ds(0, 16)][0]
        _acc_setup(zeros_hbm, acc, s)

        def body(j, carry):
            pltpu.sync_copy(onesv, acc.at[dstv.at[j]], add=True)
            return carry

        lax.fori_loop(0, ncho, body, 0)

        _acc_writeout(acc, out_hbm, c, s)

    return k


def _sigm(v):
    return 1.0 / (1.0 + jnp.exp(-v))


def _dot16(a, b):
    return jnp.dot(a.astype(jnp.bfloat16), b.astype(jnp.bfloat16),
                   preferred_element_type=jnp.float32)


def _lstm_body(x_ref, wi0_ref, wh0_ref, b0a_ref, b0b_ref,
               wi1_ref, wh1_ref, b1a_ref, b1b_ref, o_ref):
    bn = x_ref.shape[0]
    b0 = b0a_ref[...] + b0b_ref[...]
    b1 = b1a_ref[...] + b1b_ref[...]
    h0 = jnp.zeros((bn, H), jnp.float32)
    c0 = jnp.zeros((bn, H), jnp.float32)
    h1 = jnp.zeros((bn, H), jnp.float32)
    c1 = jnp.zeros((bn, H), jnp.float32)
    for t in range(T):
        xt = x_ref[:, 2 * t:2 * t + 2]
        g = _dot16(xt, wi0_ref[...]) + _dot16(h0, wh0_ref[...]) + b0
        i = _sigm(g[:, 0 * H:1 * H])
        f = _sigm(g[:, 1 * H:2 * H])
        gg = jnp.tanh(g[:, 2 * H:3 * H])
        o = _sigm(g[:, 3 * H:4 * H])
        c0 = f * c0 + i * gg
        h0 = o * jnp.tanh(c0)
        g = _dot16(h0, wi1_ref[...]) + _dot16(h1, wh1_ref[...]) + b1
        i = _sigm(g[:, 0 * H:1 * H])
        f = _sigm(g[:, 1 * H:2 * H])
        gg = jnp.tanh(g[:, 2 * H:3 * H])
        o = _sigm(g[:, 3 * H:4 * H])
        c1 = f * c1 + i * gg
        h1 = o * jnp.tanh(c1)
    o_ref[...] = h1


BN_LSTM = 1000
BN_MM = 2000


@functools.cache
def _lstm_call():
    full = lambda shape: pl.BlockSpec(shape, lambda i: (0,) * len(shape))
    return pl.pallas_call(
        _lstm_body,
        grid=(N // BN_LSTM,),
        in_specs=[
            pl.BlockSpec((BN_LSTM, T * IN_DIM), lambda i: (i, 0)),
            full((IN_DIM, 4 * H)), full((H, 4 * H)),
            full((1, 4 * H)), full((1, 4 * H)),
            full((H, 4 * H)), full((H, 4 * H)),
            full((1, 4 * H)), full((1, 4 * H)),
        ],
        out_specs=pl.BlockSpec((BN_LSTM, H), lambda i: (i, 0)),
        out_shape=jax.ShapeDtypeStruct((N, H), jnp.float32),
    )


def _mm_first_body(h_ref, w_ref, deg_ref, y_ref, dinv_ref):
    dv = lax.rsqrt(deg_ref[:, 0:1] + 1.0)
    dinv_ref[...] = jnp.broadcast_to(dv, dinv_ref.shape)
    y_ref[...] = (jnp.dot(h_ref[...], w_ref[...],
                          preferred_element_type=jnp.float32) * dv)


@functools.cache
def _mm_first_call():
    return pl.pallas_call(
        _mm_first_body,
        grid=(N // BN_MM,),
        in_specs=[
            pl.BlockSpec((BN_MM, H), lambda i: (i, 0)),
            pl.BlockSpec((H, H), lambda i: (0, 0)),
            pl.BlockSpec((BN_MM, H), lambda i: (i, 0)),
        ],
        out_specs=[
            pl.BlockSpec((BN_MM, H), lambda i: (i, 0)),
            pl.BlockSpec((BN_MM, 16), lambda i: (i, 0)),
        ],
        out_shape=[
            jax.ShapeDtypeStruct((N, H), jnp.float32),
            jax.ShapeDtypeStruct((N, 16), jnp.float32),
        ],
    )


def _mm_mid_body(agg_ref, y_ref, dinv_ref, b_ref, w_ref, o_ref):
    dv = dinv_ref[:, 0:1]
    pre = (agg_ref[...] + y_ref[...]) * dv + b_ref[...]
    hrelu = jnp.maximum(pre, 0.0)
    o_ref[...] = (jnp.dot(hrelu, w_ref[...],
                          preferred_element_type=jnp.float32) * dv)


@functools.cache
def _mm_mid_call():
    return pl.pallas_call(
        _mm_mid_body,
        grid=(N // BN_MM,),
        in_specs=[
            pl.BlockSpec((BN_MM, H), lambda i: (i, 0)),
            pl.BlockSpec((BN_MM, H), lambda i: (i, 0)),
            pl.BlockSpec((BN_MM, 16), lambda i: (i, 0)),
            pl.BlockSpec((1, H), lambda i: (0, 0)),
            pl.BlockSpec((H, H), lambda i: (0, 0)),
        ],
        out_specs=pl.BlockSpec((BN_MM, H), lambda i: (i, 0)),
        out_shape=jax.ShapeDtypeStruct((N, H), jnp.float32),
    )


def _mm_scale_body(agg_ref, y_ref, dinv_ref, b_ref, o_ref):
    dv = dinv_ref[:, 0:1]
    pre = (agg_ref[...] + y_ref[...]) * dv + b_ref[...]
    o_ref[...] = jnp.maximum(pre, 0.0) * dv


@functools.cache
def _mm_scale_call():
    return pl.pallas_call(
        _mm_scale_body,
        grid=(N // BN_MM,),
        in_specs=[
            pl.BlockSpec((BN_MM, H), lambda i: (i, 0)),
            pl.BlockSpec((BN_MM, H), lambda i: (i, 0)),
            pl.BlockSpec((BN_MM, 16), lambda i: (i, 0)),
            pl.BlockSpec((1, H), lambda i: (0, 0)),
        ],
        out_specs=pl.BlockSpec((BN_MM, H), lambda i: (i, 0)),
        out_shape=jax.ShapeDtypeStruct((N, H), jnp.float32),
    )


def _final_body(agg_ref, u_ref, dinv_ref, w_ref, b_ref, o_ref):
    du = (agg_ref[...] + u_ref[...]) * dinv_ref[:, 0:1]
    o_ref[...] = (jnp.dot(du, w_ref[...],
                          preferred_element_type=jnp.float32) + b_ref[...])


@functools.cache
def _final_call():
    return pl.pallas_call(
        _final_body,
        grid=(N // BN_MM,),
        in_specs=[
            pl.BlockSpec((BN_MM, H), lambda i: (i, 0)),
            pl.BlockSpec((BN_MM, H), lambda i: (i, 0)),
            pl.BlockSpec((BN_MM, 16), lambda i: (i, 0)),
            pl.BlockSpec((H, 16), lambda i: (0, 0)),
            pl.BlockSpec((1, 16), lambda i: (0, 0)),
        ],
        out_specs=pl.BlockSpec((BN_MM, 16), lambda i: (i, 0)),
        out_shape=jax.ShapeDtypeStruct((N, 16), jnp.float32),
    )


def kernel(x, edge_index, W_ih0, W_hh0, b_ih0, b_hh0,
           W_ih1, W_hh1, b_ih1, b_hh1, Wg1, bg1, Wg2, bg2, Wg3, bg3):
    x24 = x.reshape(N, T * IN_DIM)
    src2 = edge_index[0].reshape(NS, EPS)
    dst2 = edge_index[1].reshape(NS, EPS)
    onesK = jnp.ones((K, H), jnp.float32)
    zacc = jnp.zeros((ACC_R, H), jnp.float32)

    csrc, cdst = _compact_kernel()(src2, dst2)

    h = _lstm_call()(
        x24, W_ih0.T, W_hh0.T, b_ih0.reshape(1, -1), b_hh0.reshape(1, -1),
        W_ih1.T, W_hh1.T, b_ih1.reshape(1, -1), b_hh1.reshape(1, -1))

    deg = _deg_kernel()(cdst, onesK, zacc).reshape(N, H)
    y1, dinv16 = _mm_first_call()(h, Wg1, deg)

    agg1 = _edge_agg()(csrc, cdst, y1, zacc).reshape(N, H)
    y2 = _mm_mid_call()(agg1, y1, dinv16, bg1.reshape(1, H), Wg2)

    agg2 = _edge_agg()(csrc, cdst, y2, zacc).reshape(N, H)
    u3 = _mm_scale_call()(agg2, y2, dinv16, bg2.reshape(1, H))

    agg3 = _edge_agg()(csrc, cdst, u3, zacc).reshape(N, H)
    wg3p = jnp.zeros((H, 16), jnp.float32).at[:, :OUT_DIM].set(Wg3)
    bg3p = jnp.zeros((1, 16), jnp.float32).at[:, :OUT_DIM].set(bg3.reshape(1, -1))
    out16 = _final_call()(agg3, u3, dinv16, wg3p, bg3p)
    return out16[:, :OUT_DIM]
